# baseline (speedup 1.0000x reference)
import functools

import jax
import jax.numpy as jnp
from jax import lax
from jax.experimental import pallas as pl
from jax.experimental.pallas import tpu as pltpu
from jax.experimental.pallas import tpu_sc as plsc

B = 1024
D = 128
V = 100000
VBLK = 2000
NBLK = V // VBLK


def _score_blockmax_body(e_ref, v_ref, idx_ref, val_ref):
    j = pl.program_id(0)
    e = e_ref[...]
    v = v_ref[...]
    mm = lax.dot_general(
        v, e, (((1,), (1,)), ((), ())), preferred_element_type=jnp.float32
    )
    vsq = lax.dot_general(
        v * v, jnp.ones((D, 1), jnp.float32),
        (((1,), (0,)), ((), ())), preferred_element_type=jnp.float32,
        precision=lax.Precision.HIGHEST,
    )
    scores = mm - 0.5 * vsq

    val_ref[...] = jnp.max(scores, axis=0).reshape(1, 1, B)
    idx_ref[...] = (jnp.argmax(scores, axis=0).astype(jnp.int32)
                    .reshape(1, 1, B) + j * VBLK)


def _merge_body(val_ref, idx_ref, ids_ref):
    vals = val_ref[...].reshape(NBLK, B)
    idxs = idx_ref[...].reshape(NBLK, B)
    m = jnp.max(vals, axis=0).reshape(1, B)
    eq = vals == m
    ids_ref[...] = jnp.min(jnp.where(eq, idxs, V), axis=0).reshape(1, B)


def _argmax_scores(embeddings, vocab_embeddings):
    idxs, vals = pl.pallas_call(
        _score_blockmax_body,
        grid=(NBLK,),
        in_specs=[
            pl.BlockSpec((B, D), lambda j: (0, 0)),
            pl.BlockSpec((VBLK, D), lambda j: (j, 0)),
        ],
        out_specs=[
            pl.BlockSpec((1, 1, B), lambda j: (j, 0, 0)),
            pl.BlockSpec((1, 1, B), lambda j: (j, 0, 0)),
        ],
        out_shape=[
            jax.ShapeDtypeStruct((NBLK, 1, B), jnp.int32),
            jax.ShapeDtypeStruct((NBLK, 1, B), jnp.float32),
        ],
        compiler_params=pltpu.CompilerParams(
            dimension_semantics=("parallel",),
        ),
    )(embeddings, vocab_embeddings)

    ids = pl.pallas_call(
        _merge_body,
        in_specs=[
            pl.BlockSpec((NBLK, 1, B), lambda: (0, 0, 0)),
            pl.BlockSpec((NBLK, 1, B), lambda: (0, 0, 0)),
        ],
        out_specs=pl.BlockSpec((1, B), lambda: (0, 0)),
        out_shape=jax.ShapeDtypeStruct((1, B), jnp.int32),
    )(vals, idxs)
    return ids.reshape(B)


def _make_sc_gather():
    info = plsc.get_sparse_core_info()
    nw = info.num_cores * info.num_subcores
    b_per_w = B // nw
    mesh = plsc.VectorSubcoreMesh(core_axis_name="c", subcore_axis_name="s")

    @functools.partial(
        pl.kernel,
        mesh=mesh,
        out_type=jax.ShapeDtypeStruct((B, D), jnp.float32),
        scratch_types=[
            pltpu.VMEM((b_per_w,), jnp.int32),
            pltpu.VMEM((b_per_w, D), jnp.float32),
            pltpu.SemaphoreType.DMA,
        ],
    )
    def gather_rows(table_hbm, idx_hbm, out_hbm, idx_v, rows_v, sem):
        wid = lax.axis_index("s") * info.num_cores + lax.axis_index("c")
        base = wid * b_per_w
        pltpu.sync_copy(idx_hbm.at[pl.ds(base, b_per_w)], idx_v)
        pltpu.async_copy(table_hbm.at[idx_v], rows_v, sem).wait()
        pltpu.sync_copy(rows_v, out_hbm.at[pl.ds(base, b_per_w)])

    return gather_rows


@jax.jit
def kernel(embeddings, vocab_embeddings):
    token_ids = _argmax_scores(embeddings, vocab_embeddings)
    gather = _make_sc_gather()
    quantized = gather(vocab_embeddings, token_ids)
    return token_ids, quantized


# speedup vs baseline: 1.0401x; 1.0401x over previous
import functools

import jax
import jax.numpy as jnp
from jax import lax
from jax.experimental import pallas as pl
from jax.experimental.pallas import tpu as pltpu
from jax.experimental.pallas import tpu_sc as plsc

B = 1024
D = 128
V = 100000
VBLK = 4000
NBLK = V // VBLK


def _score_argmax_body(e_ref, v_ref, idx_ref, val_ref):
    j = pl.program_id(0)
    e = e_ref[...]
    v = v_ref[...]
    mm = lax.dot_general(
        v, e, (((1,), (1,)), ((), ())), preferred_element_type=jnp.float32
    )
    vsq = lax.dot_general(
        v * v, jnp.ones((D, 1), jnp.float32),
        (((1,), (0,)), ((), ())), preferred_element_type=jnp.float32,
        precision=lax.Precision.HIGHEST,
    )
    scores = mm - 0.5 * vsq

    m = jnp.max(scores, axis=0).reshape(1, B)
    a = (jnp.argmax(scores, axis=0).astype(jnp.int32).reshape(1, B)
         + j * VBLK)

    @pl.when(j == 0)
    def _():
        val_ref[...] = m
        idx_ref[...] = a

    @pl.when(j > 0)
    def _():
        cur = val_ref[...]
        better = m > cur
        val_ref[...] = jnp.where(better, m, cur)
        idx_ref[...] = jnp.where(better, a, idx_ref[...])


def _argmax_scores(embeddings, vocab_embeddings):
    ids, _ = pl.pallas_call(
        _score_argmax_body,
        grid=(NBLK,),
        in_specs=[
            pl.BlockSpec((B, D), lambda j: (0, 0)),
            pl.BlockSpec((VBLK, D), lambda j: (j, 0)),
        ],
        out_specs=[
            pl.BlockSpec((1, B), lambda j: (0, 0)),
            pl.BlockSpec((1, B), lambda j: (0, 0)),
        ],
        out_shape=[
            jax.ShapeDtypeStruct((1, B), jnp.int32),
            jax.ShapeDtypeStruct((1, B), jnp.float32),
        ],
    )(embeddings, vocab_embeddings)
    return ids.reshape(B)


def _make_sc_gather():
    info = plsc.get_sparse_core_info()
    nw = info.num_cores * info.num_subcores
    b_per_w = B // nw
    mesh = plsc.VectorSubcoreMesh(core_axis_name="c", subcore_axis_name="s")

    @functools.partial(
        pl.kernel,
        mesh=mesh,
        out_type=jax.ShapeDtypeStruct((B, D), jnp.float32),
        scratch_types=[
            pltpu.VMEM((b_per_w,), jnp.int32),
            pltpu.VMEM((b_per_w, D), jnp.float32),
            pltpu.SemaphoreType.DMA,
        ],
    )
    def gather_rows(table_hbm, idx_hbm, out_hbm, idx_v, rows_v, sem):
        wid = lax.axis_index("s") * info.num_cores + lax.axis_index("c")
        base = wid * b_per_w
        pltpu.sync_copy(idx_hbm.at[pl.ds(base, b_per_w)], idx_v)
        pltpu.async_copy(table_hbm.at[idx_v], rows_v, sem).wait()
        pltpu.sync_copy(rows_v, out_hbm.at[pl.ds(base, b_per_w)])

    return gather_rows


@jax.jit
def kernel(embeddings, vocab_embeddings):
    token_ids = _argmax_scores(embeddings, vocab_embeddings)
    gather = _make_sc_gather()
    quantized = gather(vocab_embeddings, token_ids)
    return token_ids, quantized


# speedup vs baseline: 1.0407x; 1.0006x over previous
import functools

import jax
import jax.numpy as jnp
from jax import lax
from jax.experimental import pallas as pl
from jax.experimental.pallas import tpu as pltpu
from jax.experimental.pallas import tpu_sc as plsc

B = 1024
D = 128
V = 100000
VBLK = 4000
NBLK = V // VBLK


def _score_argmax_body(e_ref, v_ref, idx_ref, val_ref):
    j = pl.program_id(0)
    e = e_ref[...]
    v = v_ref[...]
    mm = lax.dot_general(
        v, e, (((1,), (1,)), ((), ())), preferred_element_type=jnp.float32
    )
    vsq = lax.dot_general(
        v * v, jnp.ones((D, 1), jnp.float32),
        (((1,), (0,)), ((), ())), preferred_element_type=jnp.float32,
        precision=lax.Precision.HIGHEST,
    )
    scores = mm - 0.5 * vsq

    m = jnp.max(scores, axis=0).reshape(1, B)
    a = (jnp.argmax(scores, axis=0).astype(jnp.int32).reshape(1, B)
         + j * VBLK)

    @pl.when(j == 0)
    def _():
        val_ref[...] = m
        idx_ref[...] = a

    @pl.when(j > 0)
    def _():
        cur = val_ref[...]
        better = m > cur
        val_ref[...] = jnp.where(better, m, cur)
        idx_ref[...] = jnp.where(better, a, idx_ref[...])


def _argmax_scores(embeddings, vocab_embeddings):
    ids, _ = pl.pallas_call(
        _score_argmax_body,
        grid=(NBLK,),
        in_specs=[
            pl.BlockSpec((B, D), lambda j: (0, 0)),
            pl.BlockSpec((VBLK, D), lambda j: (j, 0)),
        ],
        out_specs=[
            pl.BlockSpec((1, B), lambda j: (0, 0)),
            pl.BlockSpec((1, B), lambda j: (0, 0)),
        ],
        out_shape=[
            jax.ShapeDtypeStruct((1, B), jnp.int32),
            jax.ShapeDtypeStruct((1, B), jnp.float32),
        ],
    )(embeddings, vocab_embeddings)
    return ids


def _make_sc_gather():
    info = plsc.get_sparse_core_info()
    nw = info.num_cores * info.num_subcores
    b_per_w = B // nw
    mesh = plsc.VectorSubcoreMesh(core_axis_name="c", subcore_axis_name="s")

    @functools.partial(
        pl.kernel,
        mesh=mesh,
        out_type=jax.ShapeDtypeStruct((B, D), jnp.float32),
        scratch_types=[
            pltpu.VMEM((b_per_w,), jnp.int32),
            pltpu.VMEM((b_per_w, D), jnp.float32),
            pltpu.SemaphoreType.DMA,
        ],
    )
    def gather_rows(table_hbm, idx_hbm, out_hbm, idx_v, rows_v, sem):
        wid = lax.axis_index("s") * info.num_cores + lax.axis_index("c")
        base = wid * b_per_w
        pltpu.sync_copy(idx_hbm.at[0, pl.ds(base, b_per_w)], idx_v)
        pltpu.async_copy(table_hbm.at[idx_v], rows_v, sem).wait()
        pltpu.sync_copy(rows_v, out_hbm.at[pl.ds(base, b_per_w)])

    return gather_rows


@jax.jit
def kernel(embeddings, vocab_embeddings):
    ids2d = _argmax_scores(embeddings, vocab_embeddings)
    gather = _make_sc_gather()
    quantized = gather(vocab_embeddings, ids2d)
    return ids2d.reshape(B), quantized
